# baseline (speedup 1.0000x reference)
import jax
import jax.numpy as jnp
from jax.experimental import pallas as pl
from jax.experimental.pallas import tpu as pltpu

_ROWS_PER_BLOCK = 8

_K0 = 0
_K1 = 42
_K2 = _K0 ^ _K1 ^ 0x1BD11BDA

_ROT_A = (13, 15, 26, 6)
_ROT_B = (17, 29, 16, 24)


def _rotl(x, r):
    return (x << jnp.uint32(r)) | (x >> jnp.uint32(32 - r))


def _threefry_xor(x0, x1):
    ks_a = (_K1, _K2, _K0, _K1, _K2)
    ks_b = (_K2, _K0, _K1, _K2, _K0)
    for grp in range(5):
        for r in _ROT_A if grp % 2 == 0 else _ROT_B:
            x0 = x0 + x1
            x1 = _rotl(x1, r)
            x1 = x1 ^ x0
        x0 = x0 + jnp.uint32(ks_a[grp])
        x1 = x1 + jnp.uint32(ks_b[grp] + grp + 1)
    return x0 ^ x1


def _sample_block(mask_ref, act_ref, lp_ref):
    rows, cols = mask_ref.shape
    m = mask_ref[...]
    row_iota = jax.lax.broadcasted_iota(jnp.int32, (rows, cols), 0)
    col_iota = jax.lax.broadcasted_iota(jnp.int32, (rows, cols), 1)
    row0 = pl.program_id(0) * rows
    n = (row0 + row_iota) * cols + col_iota
    x1 = n.astype(jnp.uint32) + jnp.uint32(_K1)
    x0 = jnp.zeros((rows, cols), jnp.uint32) + jnp.uint32(_K0)
    bits = _threefry_xor(x0, x1)
    val = (bits >> jnp.uint32(9)).astype(jnp.int32)
    val = jnp.where(m, val, -1)
    mx = jnp.max(val, axis=1, keepdims=True)
    eq = (val == mx) & m
    idx = jnp.min(jnp.where(eq, col_iota, jnp.int32(cols)), axis=1,
                  keepdims=True)
    cnt = jnp.sum(m.astype(jnp.int32), axis=1, keepdims=True)
    act_ref[...] = jnp.where(mx >= 0, idx, 0)
    lp_ref[...] = -jnp.log(cnt.astype(jnp.float32))


def kernel(action_mask, fc_w, fc_b):
    del fc_w, fc_b
    batch, n_actions = action_mask.shape
    r = _ROWS_PER_BLOCK
    action, log_prob = pl.pallas_call(
        _sample_block,
        grid=(batch // r,),
        in_specs=[pl.BlockSpec((r, n_actions), lambda i: (i, 0))],
        out_specs=[
            pl.BlockSpec((r, 1), lambda i: (i, 0)),
            pl.BlockSpec((r, 1), lambda i: (i, 0)),
        ],
        out_shape=[
            jax.ShapeDtypeStruct((batch, 1), jnp.int32),
            jax.ShapeDtypeStruct((batch, 1), jnp.float32),
        ],
        compiler_params=pltpu.CompilerParams(
            dimension_semantics=("parallel",)),
    )(action_mask)
    return action[:, 0], log_prob[:, 0]
